# baseline (speedup 1.0000x reference)
import functools

import jax
import jax.numpy as jnp
from jax import lax
from jax.experimental import pallas as pl
from jax.experimental.pallas import tpu as pltpu
from jax.experimental.pallas import tpu_sc as plsc

NUM_EMB = 1000000
DIM = 32
BATCH = 4096
NUM_IDX = 50

NC = 2
NS = 16
NW = NC * NS
ROWS_PER_W = BATCH // NW
GROUP = 8
N_GROUPS = ROWS_PER_W // GROUP
NBUF = 2

TCBR = 4000


def _pad_block(w_ref, o_ref):
    o_ref[...] = jnp.pad(w_ref[...], ((0, 0), (0, 128 - DIM)))


_pad = pl.pallas_call(
    _pad_block,
    grid=(NUM_EMB // TCBR,),
    in_specs=[pl.BlockSpec((TCBR, DIM), lambda i: (i, 0))],
    out_specs=pl.BlockSpec((TCBR, 128), lambda i: (i, 0)),
    out_shape=jax.ShapeDtypeStruct((NUM_EMB, 128), jnp.float32),
)


@functools.partial(
    pl.kernel,
    mesh=plsc.VectorSubcoreMesh(core_axis_name="c", subcore_axis_name="s"),
    out_type=jax.ShapeDtypeStruct((BATCH, NUM_IDX, DIM), jnp.float32),
    scratch_types=[
        pltpu.VMEM((ROWS_PER_W, NUM_IDX), jnp.int32),
        pltpu.VMEM((NBUF, GROUP, NUM_IDX, 128), jnp.float32),
    ] + [pltpu.SemaphoreType.DMA] * (2 * NBUF),
    compiler_params=pltpu.CompilerParams(use_tc_tiling_on_sc=False),
)
def _gather(table_hbm, idx_hbm, out_hbm, idx_v, rows_v, *sems):
    gsems, wsems = sems[:NBUF], sems[NBUF:]
    wid = lax.axis_index("s") * NC + lax.axis_index("c")
    row0 = wid * ROWS_PER_W
    pltpu.sync_copy(idx_hbm.at[pl.ds(row0, ROWS_PER_W), :], idx_v)
    wr = [None] * N_GROUPS
    for g in range(N_GROUPS):
        b = g % NBUF
        if g >= NBUF:
            wr[g - NBUF].wait()
        cps = [
            pltpu.async_copy(
                table_hbm.at[idx_v.at[g * GROUP + j]],
                rows_v.at[b, j], gsems[b])
            for j in range(GROUP)
        ]
        for cp in cps:
            cp.wait()
        wr[g] = pltpu.async_copy(
            rows_v.at[b, :, :, pl.ds(0, DIM)],
            out_hbm.at[pl.ds(row0 + g * GROUP, GROUP)],
            wsems[b])
    for g in range(N_GROUPS - NBUF, N_GROUPS):
        wr[g].wait()


def kernel(index, weight):
    return _gather(_pad(weight), index.astype(jnp.int32))


# speedup vs baseline: 1.3083x; 1.3083x over previous
import functools

import jax
import jax.numpy as jnp
from jax import lax
from jax.experimental import pallas as pl
from jax.experimental.pallas import tpu as pltpu
from jax.experimental.pallas import tpu_sc as plsc

NUM_EMB = 1000000
DIM = 32
BATCH = 4096
NUM_IDX = 50

NC = 2
NS = 16
NW = NC * NS
ROWS_PER_W = BATCH // NW
GROUP = 16
N_GROUPS = ROWS_PER_W // GROUP
NBUF = 2


@functools.partial(
    pl.kernel,
    mesh=plsc.VectorSubcoreMesh(core_axis_name="c", subcore_axis_name="s"),
    out_type=jax.ShapeDtypeStruct((BATCH, NUM_IDX, DIM), jnp.float32),
    scratch_types=[
        pltpu.VMEM((ROWS_PER_W, NUM_IDX), jnp.int32),
        pltpu.VMEM((NBUF, GROUP, NUM_IDX, DIM), jnp.float32),
    ] + [pltpu.SemaphoreType.DMA] * (2 * NBUF),
    compiler_params=pltpu.CompilerParams(use_tc_tiling_on_sc=False),
)
def _gather(table_hbm, idx_hbm, out_hbm, idx_v, rows_v, *sems):
    gsems, wsems = sems[:NBUF], sems[NBUF:]
    wid = lax.axis_index("s") * NC + lax.axis_index("c")
    row0 = wid * ROWS_PER_W
    pltpu.sync_copy(idx_hbm.at[pl.ds(row0, ROWS_PER_W), :], idx_v)
    wr = [None] * N_GROUPS
    for g in range(N_GROUPS):
        b = g % NBUF
        if g >= NBUF:
            wr[g - NBUF].wait()
        cps = [
            pltpu.async_copy(
                table_hbm.at[idx_v.at[g * GROUP + j]], rows_v.at[b, j],
                gsems[b])
            for j in range(GROUP)
        ]
        for cp in cps:
            cp.wait()
        wr[g] = pltpu.async_copy(
            rows_v.at[b], out_hbm.at[pl.ds(row0 + g * GROUP, GROUP)],
            wsems[b])
    for g in range(N_GROUPS - NBUF, N_GROUPS):
        wr[g].wait()


def kernel(index, weight):
    return _gather(weight, index.astype(jnp.int32))
